# baseline (speedup 1.0000x reference)
import jax
import jax.numpy as jnp
from jax import lax
from jax.experimental import pallas as pl
from jax.experimental.pallas import tpu as pltpu
from jax.experimental.pallas import tpu_sc as plsc

NC = 2
NS = 16
LANES = 16
BLK = 128
RC = 64
NBUF = 4
GLK = 2
IDXC = 32


def _zero_2d(buf, rows, cols):
    z = jnp.zeros((LANES,), jnp.float32)

    def body(r, _):
        for k4 in range(cols // LANES):
            buf[r, pl.ds(LANES * k4, LANES)] = z
        return 0

    lax.fori_loop(0, rows, body, 0)


def _fill_1d(buf, n, val):
    v = jnp.full((LANES,), val, jnp.float32)

    def body(i, _):
        buf[pl.ds(LANES * i, LANES)] = v
        return 0

    lax.fori_loop(0, n // LANES, body, 0)


def _scale_rows(nbuf, dbuf, rows, cols):
    def body(g, _):
        dvec = dbuf[pl.ds(LANES * g, LANES)]
        nvec = 1.0 / jnp.maximum(dvec, 1.0)
        for i in range(LANES):
            n = nvec[i]
            r = LANES * g + i
            for k4 in range(cols // LANES):
                sl = pl.ds(LANES * k4, LANES)
                nbuf[r, sl] = nbuf[r, sl] * n
        return 0

    lax.fori_loop(0, rows // LANES, body, 0)


def _make_kernel(n_pad, dcol, k_blocks):
    rp = n_pad // NS
    assert rp % RC == 0
    nrc = rp // RC
    assert k_blocks % IDXC == 0
    nic = k_blocks // IDXC

    mesh = plsc.VectorSubcoreMesh(core_axis_name="c", subcore_axis_name="s")

    def body(hc_ref, srcb_ref, dstb_ref, rst_ref, bsrc_ref,
             A, B, degd, degs, src_v, dst_v, rowb, nbuf, dbuf, ones_v,
             gsem, ssem, dsem):
        c = lax.axis_index("c")
        s = lax.axis_index("s")
        z0 = s * rp

        def edge_pass(gat_sh, sct_sh, g_is_src, with_deg):
            idx_g = src_v if g_is_src else dst_v
            idx_s = dst_v if g_is_src else src_v

            def chunk(ci, _):
                pltpu.sync_copy(srcb_ref.at[s, pl.ds(ci * IDXC, IDXC)],
                                src_v)
                pltpu.sync_copy(dstb_ref.at[s, pl.ds(ci * IDXC, IDXC)],
                                dst_v)
                for t in range(GLK):
                    pltpu.async_copy(gat_sh.at[idx_g.at[t]], rowb.at[t],
                                     gsem)

                def blk(jj, _):
                    b = lax.rem(jj, NBUF)
                    pltpu.make_async_copy(
                        gat_sh.at[idx_g.at[jj]], rowb.at[b], gsem).wait()

                    @pl.when(jj >= GLK)
                    def _():
                        pltpu.make_async_copy(
                            rowb.at[b], sct_sh.at[idx_s.at[jj]], ssem).wait()
                        if with_deg:
                            pltpu.make_async_copy(
                                ones_v, degd.at[idx_s.at[jj]], dsem).wait()
                            pltpu.make_async_copy(
                                ones_v, degs.at[idx_g.at[jj]], dsem).wait()

                    @pl.when(jj + GLK < IDXC)
                    def _():
                        pltpu.async_copy(
                            gat_sh.at[idx_g.at[jj + GLK]],
                            rowb.at[lax.rem(jj + GLK, NBUF)], gsem)

                    pltpu.async_copy(rowb.at[b], sct_sh.at[idx_s.at[jj]],
                                     ssem, add=True)
                    if with_deg:
                        pltpu.async_copy(ones_v, degd.at[idx_s.at[jj]],
                                         dsem, add=True)
                        pltpu.async_copy(ones_v, degs.at[idx_g.at[jj]],
                                         dsem, add=True)
                    return 0

                lax.fori_loop(0, IDXC, blk, 0)
                for t in range(GLK):
                    pltpu.make_async_copy(rowb.at[0], sct_sh.at[idx_s.at[0]],
                                          ssem).wait()
                    if with_deg:
                        pltpu.make_async_copy(ones_v, degd.at[idx_s.at[0]],
                                              dsem).wait()
                        pltpu.make_async_copy(ones_v, degs.at[idx_g.at[0]],
                                              dsem).wait()
                return 0

            lax.fori_loop(0, nic, chunk, 0)

        _zero_2d(nbuf, RC, dcol)
        _fill_1d(dbuf, RC, 0.0)
        _fill_1d(ones_v, BLK, 1.0)

        def zchunk(cr, _):
            r0 = z0 + cr * RC
            pltpu.sync_copy(nbuf, B.at[pl.ds(r0, RC)])
            pltpu.sync_copy(dbuf, degd.at[pl.ds(r0, RC)])
            pltpu.sync_copy(dbuf, degs.at[pl.ds(r0, RC)])
            return 0

        lax.fori_loop(0, nrc, zchunk, 0)

        def hchunk(cr, _):
            r0 = z0 + cr * RC
            pltpu.sync_copy(hc_ref.at[c, pl.ds(r0, RC)], nbuf)
            pltpu.sync_copy(nbuf, A.at[pl.ds(r0, RC)])
            return 0

        lax.fori_loop(0, nrc, hchunk, 0)

        plsc.subcore_barrier()

        edge_pass(A, B, True, True)

        plsc.subcore_barrier()

        def nchunk(cr, _):
            r0 = z0 + cr * RC
            pltpu.sync_copy(degd.at[pl.ds(r0, RC)], dbuf)
            pltpu.sync_copy(B.at[pl.ds(r0, RC)], nbuf)
            _scale_rows(nbuf, dbuf, RC, dcol)
            pltpu.sync_copy(nbuf, B.at[pl.ds(r0, RC)])
            pltpu.sync_copy(nbuf, rst_ref.at[c, pl.ds(r0, RC)])
            return 0

        lax.fori_loop(0, nrc, nchunk, 0)
        _zero_2d(nbuf, RC, dcol)

        def azchunk(cr, _):
            pltpu.sync_copy(nbuf, A.at[pl.ds(z0 + cr * RC, RC)])
            return 0

        lax.fori_loop(0, nrc, azchunk, 0)

        plsc.subcore_barrier()

        edge_pass(B, A, False, False)

        plsc.subcore_barrier()

        def bchunk(cr, _):
            r0 = z0 + cr * RC
            pltpu.sync_copy(degs.at[pl.ds(r0, RC)], dbuf)
            pltpu.sync_copy(A.at[pl.ds(r0, RC)], nbuf)
            _scale_rows(nbuf, dbuf, RC, dcol)
            pltpu.sync_copy(nbuf, bsrc_ref.at[c, pl.ds(r0, RC)])
            return 0

        lax.fori_loop(0, nrc, bchunk, 0)

    return pl.kernel(
        body,
        out_type=[
            jax.ShapeDtypeStruct((NC, n_pad, dcol), jnp.float32),
            jax.ShapeDtypeStruct((NC, n_pad, dcol), jnp.float32),
        ],
        mesh=mesh,
        compiler_params=pltpu.CompilerParams(use_tc_tiling_on_sc=False),
        scratch_types=[
            pltpu.VMEM_SHARED((n_pad, dcol), jnp.float32),
            pltpu.VMEM_SHARED((n_pad, dcol), jnp.float32),
            pltpu.VMEM_SHARED((n_pad,), jnp.float32),
            pltpu.VMEM_SHARED((n_pad,), jnp.float32),
            pltpu.VMEM((IDXC, BLK), jnp.int32),
            pltpu.VMEM((IDXC, BLK), jnp.int32),
            pltpu.VMEM((NBUF, BLK, dcol), jnp.float32),
            pltpu.VMEM((RC, dcol), jnp.float32),
            pltpu.VMEM((RC,), jnp.float32),
            pltpu.VMEM((BLK,), jnp.float32),
            pltpu.SemaphoreType.DMA,
            pltpu.SemaphoreType.DMA,
            pltpu.SemaphoreType.DMA,
        ],
    )


def kernel(h_src, h_dst, edge_index):
    n_src, d = h_src.shape
    n_dst = h_dst.shape[0]
    assert n_src == n_dst
    e = edge_index.shape[1]
    dcol = d // NC

    k_blocks = -(-e // (NS * BLK * IDXC)) * IDXC
    e_pad = NS * k_blocks * BLK
    n_pad = ((n_src + NS * RC - 1) // (NS * RC)) * (NS * RC)

    src = edge_index[0].astype(jnp.int32)
    dst = edge_index[1].astype(jnp.int32)
    pad = e_pad - e
    if pad:
        trash = jnp.full((pad,), n_src, jnp.int32)
        src = jnp.concatenate([src, trash])
        dst = jnp.concatenate([dst, trash])
    srcb = src.reshape(NS, k_blocks, BLK)
    dstb = dst.reshape(NS, k_blocks, BLK)

    hc = h_src.reshape(n_src, NC, dcol).transpose(1, 0, 2)
    hc = jnp.pad(hc, ((0, 0), (0, n_pad - n_src), (0, 0)))

    rst_o, bsrc_o = _make_kernel(n_pad, dcol, k_blocks)(hc, srcb, dstb)

    rst = jnp.concatenate([rst_o[i] for i in range(NC)], axis=-1)[:n_dst]
    bsrc = jnp.concatenate([bsrc_o[i] for i in range(NC)], axis=-1)[:n_src]
    return (bsrc, rst)


# speedup vs baseline: 1.1556x; 1.1556x over previous
import jax
import jax.numpy as jnp
from jax import lax
from jax.experimental import pallas as pl
from jax.experimental.pallas import tpu as pltpu
from jax.experimental.pallas import tpu_sc as plsc

NC = 2
NS = 16
LANES = 16
BLK = 128
RC = 64
NBUF = 5
GLOOK = 3
SLAG = 2


def _zero_2d(buf, rows, cols):
    z = jnp.zeros((LANES,), jnp.float32)

    def body(r, _):
        for k4 in range(cols // LANES):
            buf[r, pl.ds(LANES * k4, LANES)] = z
        return 0

    lax.fori_loop(0, rows, body, 0)


def _fill_1d(buf, n, val):
    v = jnp.full((LANES,), val, jnp.float32)

    def body(i, _):
        buf[pl.ds(LANES * i, LANES)] = v
        return 0

    lax.fori_loop(0, n // LANES, body, 0)


def _scale_rows(nbuf, dbuf, rows, cols):
    def body(g, _):
        dvec = dbuf[pl.ds(LANES * g, LANES)]
        nvec = 1.0 / jnp.maximum(dvec, 1.0)
        for i in range(LANES):
            n = nvec[i]
            r = LANES * g + i
            for k4 in range(cols // LANES):
                sl = pl.ds(LANES * k4, LANES)
                nbuf[r, sl] = nbuf[r, sl] * n
        return 0

    lax.fori_loop(0, rows // LANES, body, 0)


def _make_kernel(n_pad, dcol, k_blocks):
    rp = n_pad // NS
    assert rp % RC == 0
    nrc = rp // RC

    mesh = plsc.VectorSubcoreMesh(core_axis_name="c", subcore_axis_name="s")

    def body(hc_ref, srcb_ref, dstb_ref, rst_ref, bsrc_ref,
             B, degd, degs, src_v, dst_v, rowb, nbuf, dbuf, ones_v,
             gsem, ssem, dsem):
        c = lax.axis_index("c")
        s = lax.axis_index("s")
        z0 = s * rp

        def edge_pass(gat_hbm, idx_g, idx_s, with_deg):
            for t in range(GLOOK):
                pltpu.async_copy(gat_hbm.at[idx_g.at[t]], rowb.at[t], gsem)

            def blk(j, _):
                b = lax.rem(j, NBUF)
                pltpu.make_async_copy(
                    gat_hbm.at[idx_g.at[j]], rowb.at[b], gsem).wait()

                @pl.when(j >= SLAG)
                def _():
                    pltpu.make_async_copy(
                        rowb.at[b], B.at[idx_s.at[j]], ssem).wait()
                    if with_deg:
                        pltpu.make_async_copy(
                            ones_v, degd.at[idx_s.at[j]], dsem).wait()
                        pltpu.make_async_copy(
                            ones_v, degs.at[idx_g.at[j]], dsem).wait()

                @pl.when(j + GLOOK < k_blocks)
                def _():
                    pltpu.async_copy(
                        gat_hbm.at[idx_g.at[j + GLOOK]],
                        rowb.at[lax.rem(j + GLOOK, NBUF)], gsem)

                pltpu.async_copy(rowb.at[b], B.at[idx_s.at[j]], ssem,
                                 add=True)
                if with_deg:
                    pltpu.async_copy(ones_v, degd.at[idx_s.at[j]], dsem,
                                     add=True)
                    pltpu.async_copy(ones_v, degs.at[idx_g.at[j]], dsem,
                                     add=True)
                return 0

            lax.fori_loop(0, k_blocks, blk, 0)
            for t in range(SLAG):
                pltpu.make_async_copy(rowb.at[0], B.at[idx_s.at[0]],
                                      ssem).wait()
                if with_deg:
                    pltpu.make_async_copy(ones_v, degd.at[idx_s.at[0]],
                                          dsem).wait()
                    pltpu.make_async_copy(ones_v, degs.at[idx_g.at[0]],
                                          dsem).wait()

        pltpu.sync_copy(srcb_ref.at[s], src_v)
        pltpu.sync_copy(dstb_ref.at[s], dst_v)
        _zero_2d(nbuf, RC, dcol)
        _fill_1d(dbuf, RC, 0.0)
        _fill_1d(ones_v, BLK, 1.0)

        def zchunk(cr, _):
            r0 = z0 + cr * RC
            pltpu.sync_copy(nbuf, B.at[pl.ds(r0, RC)])
            pltpu.sync_copy(dbuf, degd.at[pl.ds(r0, RC)])
            pltpu.sync_copy(dbuf, degs.at[pl.ds(r0, RC)])
            return 0

        lax.fori_loop(0, nrc, zchunk, 0)

        plsc.subcore_barrier()

        edge_pass(hc_ref.at[c], src_v, dst_v, True)

        plsc.subcore_barrier()

        def nchunk(cr, _):
            r0 = z0 + cr * RC
            pltpu.sync_copy(degd.at[pl.ds(r0, RC)], dbuf)
            pltpu.sync_copy(B.at[pl.ds(r0, RC)], nbuf)
            _scale_rows(nbuf, dbuf, RC, dcol)
            pltpu.sync_copy(nbuf, rst_ref.at[c, pl.ds(r0, RC)])
            return 0

        lax.fori_loop(0, nrc, nchunk, 0)
        _zero_2d(nbuf, RC, dcol)

        def bzchunk(cr, _):
            pltpu.sync_copy(nbuf, B.at[pl.ds(z0 + cr * RC, RC)])
            return 0

        lax.fori_loop(0, nrc, bzchunk, 0)

        plsc.subcore_barrier()

        edge_pass(rst_ref.at[c], dst_v, src_v, False)

        plsc.subcore_barrier()

        def bchunk(cr, _):
            r0 = z0 + cr * RC
            pltpu.sync_copy(degs.at[pl.ds(r0, RC)], dbuf)
            pltpu.sync_copy(B.at[pl.ds(r0, RC)], nbuf)
            _scale_rows(nbuf, dbuf, RC, dcol)
            pltpu.sync_copy(nbuf, bsrc_ref.at[c, pl.ds(r0, RC)])
            return 0

        lax.fori_loop(0, nrc, bchunk, 0)

    return pl.kernel(
        body,
        out_type=[
            jax.ShapeDtypeStruct((NC, n_pad, dcol), jnp.float32),
            jax.ShapeDtypeStruct((NC, n_pad, dcol), jnp.float32),
        ],
        mesh=mesh,
        compiler_params=pltpu.CompilerParams(use_tc_tiling_on_sc=False),
        scratch_types=[
            pltpu.VMEM_SHARED((n_pad, dcol), jnp.float32),
            pltpu.VMEM_SHARED((n_pad,), jnp.float32),
            pltpu.VMEM_SHARED((n_pad,), jnp.float32),
            pltpu.VMEM((k_blocks, BLK), jnp.int32),
            pltpu.VMEM((k_blocks, BLK), jnp.int32),
            pltpu.VMEM((NBUF, BLK, dcol), jnp.float32),
            pltpu.VMEM((RC, dcol), jnp.float32),
            pltpu.VMEM((RC,), jnp.float32),
            pltpu.VMEM((BLK,), jnp.float32),
            pltpu.SemaphoreType.DMA,
            pltpu.SemaphoreType.DMA,
            pltpu.SemaphoreType.DMA,
        ],
    )


def kernel(h_src, h_dst, edge_index):
    n_src, d = h_src.shape
    n_dst = h_dst.shape[0]
    assert n_src == n_dst
    e = edge_index.shape[1]
    dcol = d // NC

    k_blocks = -(-e // (NS * BLK))
    e_pad = NS * k_blocks * BLK
    n_pad = ((n_src + NS * RC - 1) // (NS * RC)) * (NS * RC)

    src = edge_index[0].astype(jnp.int32)
    dst = edge_index[1].astype(jnp.int32)
    pad = e_pad - e
    if pad:
        trash = jnp.full((pad,), n_src, jnp.int32)
        src = jnp.concatenate([src, trash])
        dst = jnp.concatenate([dst, trash])
    srcb = src.reshape(NS, k_blocks, BLK)
    dstb = dst.reshape(NS, k_blocks, BLK)

    hc = h_src.reshape(n_src, NC, dcol).transpose(1, 0, 2)
    hc = jnp.pad(hc, ((0, 0), (0, n_pad - n_src), (0, 0)))

    rst_o, bsrc_o = _make_kernel(n_pad, dcol, k_blocks)(hc, srcb, dstb)

    rst = jnp.concatenate([rst_o[i] for i in range(NC)], axis=-1)[:n_dst]
    bsrc = jnp.concatenate([bsrc_o[i] for i in range(NC)], axis=-1)[:n_src]
    return (bsrc, rst)
